# baseline (speedup 1.0000x reference)
import functools

import jax
import jax.numpy as jnp
from jax import lax
from jax.experimental import pallas as pl
from jax.experimental.pallas import tpu as pltpu
from jax.experimental.pallas import tpu_sc as plsc

LATENT = 128
HIDDEN = 256
OUT_DIM = 1
B_GRAPHS = 500
NPG = 200

NC = 2
NS = 16
NW = NC * NS
MAX_G_PER_W = -(-B_GRAPHS // NW)
LANES = 16
NCHUNK = LATENT // LANES


def _sc_pool_body(feat_hbm, out_hbm, buf, rowbuf, sem0, sem1):
    wid = lax.axis_index("s") * NC + lax.axis_index("c")
    sems = (sem0, sem1)

    def copy_for(i):
        g = wid + i * NW
        return pltpu.make_async_copy(
            feat_hbm.at[pl.ds(g * NPG, NPG)], buf.at[i % 2], sems[i % 2])

    def start(i):
        @pl.when(wid + i * NW < B_GRAPHS)
        def _():
            copy_for(i).start()

    start(0)
    for i in range(MAX_G_PER_W):
        if i + 1 < MAX_G_PER_W:
            start(i + 1)
        g = wid + i * NW

        @pl.when(g < B_GRAPHS)
        def _process(i=i, g=g):
            copy_for(i).wait()
            slot = i % 2

            def acc_body(r, c):
                return tuple(
                    c[j] + buf[slot, r, pl.ds(j * LANES, LANES)]
                    for j in range(NCHUNK))

            zeros = tuple(
                jnp.zeros((LANES,), jnp.float32) for _ in range(NCHUNK))
            sums = plsc.parallel_loop(
                0, NPG, 1, unroll=8, carry=zeros)(acc_body)
            for j in range(NCHUNK):
                rowbuf[0, pl.ds(j * LANES, LANES)] = sums[j]
            pltpu.sync_copy(rowbuf, out_hbm.at[pl.ds(g, 1)])


def _sc_pool(feat):
    mesh = plsc.VectorSubcoreMesh(
        core_axis_name="c", subcore_axis_name="s", num_cores=NC,
        num_subcores=NS)
    return pl.kernel(
        _sc_pool_body,
        out_type=jax.ShapeDtypeStruct((B_GRAPHS, LATENT), jnp.float32),
        mesh=mesh,
        scratch_types=[
            pltpu.VMEM((2, NPG, LATENT), jnp.float32),
            pltpu.VMEM((1, LATENT), jnp.float32),
            pltpu.SemaphoreType.DMA,
            pltpu.SemaphoreType.DMA,
        ],
    )(feat)


def _mlp_kernel(pooled_ref, n_ref, w1_ref, b1_ref, w2_ref, b2_ref,
                w3_ref, b3_ref, out_ref):
    pooled = pooled_ref[...] / n_ref[...].astype(jnp.float32)
    h = jnp.maximum(
        jnp.dot(pooled, w1_ref[...], preferred_element_type=jnp.float32)
        + b1_ref[...], 0.0)
    h = jnp.maximum(
        jnp.dot(h, w2_ref[...], preferred_element_type=jnp.float32)
        + b2_ref[...], 0.0)
    out_ref[...] = (
        jnp.dot(h, w3_ref[...], preferred_element_type=jnp.float32)
        + b3_ref[...])


def _tc_mlp(pooled_sum, n_node, W1, b1, W2, b2, W3, b3):
    return pl.pallas_call(
        _mlp_kernel,
        out_shape=jax.ShapeDtypeStruct((B_GRAPHS, OUT_DIM), jnp.float32),
    )(pooled_sum, n_node.reshape(B_GRAPHS, 1), W1, b1, W2, b2, W3, b3)


@jax.jit
def kernel(feat, n_node, W1, b1, W2, b2, W3, b3):
    pooled_sum = _sc_pool(feat)
    return _tc_mlp(pooled_sum, n_node, W1, b1, W2, b2, W3, b3)


# speedup vs baseline: 1.2504x; 1.2504x over previous
import functools

import jax
import jax.numpy as jnp
from jax import lax
from jax.experimental import pallas as pl
from jax.experimental.pallas import tpu as pltpu
from jax.experimental.pallas import tpu_sc as plsc

LATENT = 128
HIDDEN = 256
OUT_DIM = 1
B_GRAPHS = 500
NPG = 200
ROW = NPG * LATENT

NC = 2
NS = 16
NW = NC * NS
LANES = 16
NCHUNK = LATENT // LANES

K_SC = 160
K_TC = B_GRAPHS - K_SC
SC_ITERS = -(-K_SC // NW)
G_BLK = 40
TC_STEPS = -(-K_TC // G_BLK)
TC_OFF = K_SC // G_BLK


def _sc_pool_body(feat_hbm, out_hbm, buf, rowbuf, sem0, sem1):
    wid = lax.axis_index("s") * NC + lax.axis_index("c")
    sems = (sem0, sem1)

    def copy_for(i):
        g = wid + i * NW
        return pltpu.make_async_copy(
            feat_hbm.at[pl.ds(g * ROW, ROW)], buf.at[i % 2], sems[i % 2])

    def start(i):
        @pl.when(wid + i * NW < K_SC)
        def _():
            copy_for(i).start()

    start(0)
    for i in range(SC_ITERS):
        if i + 1 < SC_ITERS:
            start(i + 1)
        g = wid + i * NW

        @pl.when(g < K_SC)
        def _process(i=i, g=g):
            copy_for(i).wait()
            slot = i % 2

            def acc_body(r, c):
                base = r * LATENT
                return tuple(
                    c[j] + buf[slot, pl.ds(base + j * LANES, LANES)]
                    for j in range(NCHUNK))

            zeros = tuple(
                jnp.zeros((LANES,), jnp.float32) for _ in range(NCHUNK))
            sums = plsc.parallel_loop(
                0, NPG, 1, unroll=8, carry=zeros)(acc_body)
            for j in range(NCHUNK):
                rowbuf[0, pl.ds(j * LANES, LANES)] = sums[j]
            pltpu.sync_copy(rowbuf, out_hbm.at[pl.ds(g, 1)])


def _sc_pool(feat_flat):
    mesh = plsc.VectorSubcoreMesh(
        core_axis_name="c", subcore_axis_name="s", num_cores=NC,
        num_subcores=NS)
    return pl.kernel(
        _sc_pool_body,
        out_type=jax.ShapeDtypeStruct((K_SC, LATENT), jnp.float32),
        mesh=mesh,
        scratch_types=[
            pltpu.VMEM((2, ROW), jnp.float32),
            pltpu.VMEM((1, LATENT), jnp.float32),
            pltpu.SemaphoreType.DMA,
            pltpu.SemaphoreType.DMA,
        ],
    )(feat_flat)


def _tc_pool_kernel(feat_ref, out_ref):
    out_ref[...] = jnp.sum(feat_ref[...], axis=1)


def _tc_pool(feat3):
    return pl.pallas_call(
        _tc_pool_kernel,
        grid=(TC_STEPS,),
        in_specs=[
            pl.BlockSpec((G_BLK, NPG, LATENT), lambda i: (TC_OFF + i, 0, 0)),
        ],
        out_specs=pl.BlockSpec((G_BLK, LATENT), lambda i: (i, 0)),
        out_shape=jax.ShapeDtypeStruct((K_TC, LATENT), jnp.float32),
    )(feat3)


def _mlp_kernel(sc_ref, tc_ref, n_ref, w1_ref, b1_ref, w2_ref, b2_ref,
                w3_ref, b3_ref, out_ref):
    pooled = jnp.concatenate([sc_ref[...], tc_ref[...]], axis=0)
    pooled = pooled / n_ref[...].astype(jnp.float32)
    h = jnp.maximum(
        jnp.dot(pooled, w1_ref[...], preferred_element_type=jnp.float32)
        + b1_ref[...], 0.0)
    h = jnp.maximum(
        jnp.dot(h, w2_ref[...], preferred_element_type=jnp.float32)
        + b2_ref[...], 0.0)
    out_ref[...] = (
        jnp.dot(h, w3_ref[...], preferred_element_type=jnp.float32)
        + b3_ref[...])


def _tc_mlp(sc_sums, tc_sums, n_node, W1, b1, W2, b2, W3, b3):
    return pl.pallas_call(
        _mlp_kernel,
        out_shape=jax.ShapeDtypeStruct((B_GRAPHS, OUT_DIM), jnp.float32),
    )(sc_sums, tc_sums, n_node.reshape(B_GRAPHS, 1), W1, b1, W2, b2, W3, b3)


@jax.jit
def kernel(feat, n_node, W1, b1, W2, b2, W3, b3):
    sc_sums = _sc_pool(feat.reshape(-1))
    tc_sums = _tc_pool(feat.reshape(B_GRAPHS, NPG, LATENT))
    return _tc_mlp(sc_sums, tc_sums, n_node, W1, b1, W2, b2, W3, b3)


# speedup vs baseline: 1.2525x; 1.0017x over previous
import functools

import jax
import jax.numpy as jnp
from jax import lax
from jax.experimental import pallas as pl
from jax.experimental.pallas import tpu as pltpu
from jax.experimental.pallas import tpu_sc as plsc

LATENT = 128
HIDDEN = 256
OUT_DIM = 1
B_GRAPHS = 500
NPG = 200
ROW = NPG * LATENT

NC = 2
NS = 16
NW = NC * NS
LANES = 16
NCHUNK = LATENT // LANES

K_SC = 192
K_TC = B_GRAPHS - K_SC
SC_ITERS = K_SC // NW
G_BLK = 48
TC_STEPS = -(-K_TC // G_BLK)
TC_OFF = K_SC // G_BLK


def _sc_pool_body(feat_hbm, out_hbm, buf, rowbuf, sem0, sem1):
    wid = lax.axis_index("s") * NC + lax.axis_index("c")

    def copy_for(i, slot, sem):
        g = wid + i * NW
        return pltpu.make_async_copy(
            feat_hbm.at[pl.ds(g * ROW, ROW)], buf.at[slot], sem)

    def consume(i, slot, sem):
        g = wid + i * NW
        copy_for(i, slot, sem).wait()

        def acc_body(r, c):
            base = r * LATENT
            return tuple(
                c[j] + buf[slot, pl.ds(base + j * LANES, LANES)]
                for j in range(NCHUNK))

        zeros = tuple(
            jnp.zeros((LANES,), jnp.float32) for _ in range(NCHUNK))
        sums = plsc.parallel_loop(
            0, NPG, 1, unroll=4, carry=zeros)(acc_body)
        for j in range(NCHUNK):
            rowbuf[0, pl.ds(j * LANES, LANES)] = sums[j]
        pltpu.sync_copy(rowbuf, out_hbm.at[pl.ds(g, 1)])

    copy_for(0, 0, sem0).start()

    def pair_body(k, carry):
        i0 = 2 * k
        copy_for(i0 + 1, 1, sem1).start()
        consume(i0, 0, sem0)

        @pl.when(i0 + 2 < SC_ITERS)
        def _():
            copy_for(i0 + 2, 0, sem0).start()

        consume(i0 + 1, 1, sem1)
        return carry

    lax.fori_loop(0, SC_ITERS // 2, pair_body, 0)


def _sc_pool(feat_flat):
    mesh = plsc.VectorSubcoreMesh(
        core_axis_name="c", subcore_axis_name="s", num_cores=NC,
        num_subcores=NS)
    return pl.kernel(
        _sc_pool_body,
        out_type=jax.ShapeDtypeStruct((K_SC, LATENT), jnp.float32),
        mesh=mesh,
        scratch_types=[
            pltpu.VMEM((2, ROW), jnp.float32),
            pltpu.VMEM((1, LATENT), jnp.float32),
            pltpu.SemaphoreType.DMA,
            pltpu.SemaphoreType.DMA,
        ],
    )(feat_flat)


def _tc_pool_kernel(feat_ref, out_ref):
    out_ref[...] = jnp.sum(feat_ref[...], axis=1)


def _tc_pool(feat3):
    return pl.pallas_call(
        _tc_pool_kernel,
        grid=(TC_STEPS,),
        in_specs=[
            pl.BlockSpec((G_BLK, NPG, LATENT), lambda i: (TC_OFF + i, 0, 0)),
        ],
        out_specs=pl.BlockSpec((G_BLK, LATENT), lambda i: (i, 0)),
        out_shape=jax.ShapeDtypeStruct((K_TC, LATENT), jnp.float32),
    )(feat3)


def _mlp_kernel(sc_ref, tc_ref, n_ref, w1_ref, b1_ref, w2_ref, b2_ref,
                w3_ref, b3_ref, out_ref):
    def head(pooled):
        h = jnp.maximum(
            jnp.dot(pooled, w1_ref[...], preferred_element_type=jnp.float32)
            + b1_ref[...], 0.0)
        h = jnp.maximum(
            jnp.dot(h, w2_ref[...], preferred_element_type=jnp.float32)
            + b2_ref[...], 0.0)
        return (jnp.dot(h, w3_ref[...], preferred_element_type=jnp.float32)
                + b3_ref[...])

    n = n_ref[...].astype(jnp.float32)
    out_ref[:K_SC] = head(sc_ref[...] / n[:K_SC])
    out_ref[K_SC:] = head(tc_ref[...] / n[K_SC:])


def _tc_mlp(sc_sums, tc_sums, n_node, W1, b1, W2, b2, W3, b3):
    return pl.pallas_call(
        _mlp_kernel,
        out_shape=jax.ShapeDtypeStruct((B_GRAPHS, OUT_DIM), jnp.float32),
    )(sc_sums, tc_sums, n_node.reshape(B_GRAPHS, 1), W1, b1, W2, b2, W3, b3)


@jax.jit
def kernel(feat, n_node, W1, b1, W2, b2, W3, b3):
    sc_sums = _sc_pool(feat.reshape(-1))
    tc_sums = _tc_pool(feat.reshape(B_GRAPHS, NPG, LATENT))
    return _tc_mlp(sc_sums, tc_sums, n_node, W1, b1, W2, b2, W3, b3)


# speedup vs baseline: 1.2530x; 1.0004x over previous
import functools

import jax
import jax.numpy as jnp
from jax import lax
from jax.experimental import pallas as pl
from jax.experimental.pallas import tpu as pltpu
from jax.experimental.pallas import tpu_sc as plsc

LATENT = 128
HIDDEN = 256
OUT_DIM = 1
B_GRAPHS = 500
NPG = 200
ROW = NPG * LATENT

NC = 2
NS = 16
NW = NC * NS
LANES = 16
NCHUNK = LATENT // LANES

K_SC = 192
K_TC = B_GRAPHS - K_SC
SC_ITERS = K_SC // NW
G_BLK = 48
TC_STEPS = -(-K_TC // G_BLK)
TC_OFF = K_SC // G_BLK


def _sc_pool_body(feat_hbm, out_hbm, buf, rowbuf, sem0, sem1):
    wid = lax.axis_index("s") * NC + lax.axis_index("c")

    def copy_for(i, slot, sem):
        g = wid + i * NW
        return pltpu.make_async_copy(
            feat_hbm.at[pl.ds(g * ROW, ROW)], buf.at[slot], sem)

    def consume(i, slot, sem):
        g = wid + i * NW
        copy_for(i, slot, sem).wait()

        def acc_body(r, c):
            base = r * LATENT
            return tuple(
                c[j] + buf[slot, pl.ds(base + j * LANES, LANES)]
                for j in range(NCHUNK))

        zeros = tuple(
            jnp.zeros((LANES,), jnp.float32) for _ in range(NCHUNK))
        sums = plsc.parallel_loop(
            0, NPG, 1, unroll=2, carry=zeros)(acc_body)
        for j in range(NCHUNK):
            rowbuf[0, pl.ds(j * LANES, LANES)] = sums[j]
        pltpu.sync_copy(rowbuf, out_hbm.at[pl.ds(g, 1)])

    copy_for(0, 0, sem0).start()

    def pair_body(k, carry):
        i0 = 2 * k
        copy_for(i0 + 1, 1, sem1).start()
        consume(i0, 0, sem0)

        @pl.when(i0 + 2 < SC_ITERS)
        def _():
            copy_for(i0 + 2, 0, sem0).start()

        consume(i0 + 1, 1, sem1)
        return carry

    lax.fori_loop(0, SC_ITERS // 2, pair_body, 0)


def _sc_pool(feat_flat):
    mesh = plsc.VectorSubcoreMesh(
        core_axis_name="c", subcore_axis_name="s", num_cores=NC,
        num_subcores=NS)
    return pl.kernel(
        _sc_pool_body,
        out_type=jax.ShapeDtypeStruct((K_SC, LATENT), jnp.float32),
        mesh=mesh,
        scratch_types=[
            pltpu.VMEM((2, ROW), jnp.float32),
            pltpu.VMEM((1, LATENT), jnp.float32),
            pltpu.SemaphoreType.DMA,
            pltpu.SemaphoreType.DMA,
        ],
    )(feat_flat)


def _tc_pool_kernel(feat_ref, out_ref):
    out_ref[...] = jnp.sum(feat_ref[...], axis=1)


def _tc_pool(feat3):
    return pl.pallas_call(
        _tc_pool_kernel,
        grid=(TC_STEPS,),
        in_specs=[
            pl.BlockSpec((G_BLK, NPG, LATENT), lambda i: (TC_OFF + i, 0, 0)),
        ],
        out_specs=pl.BlockSpec((G_BLK, LATENT), lambda i: (i, 0)),
        out_shape=jax.ShapeDtypeStruct((K_TC, LATENT), jnp.float32),
    )(feat3)


def _mlp_kernel(sc_ref, tc_ref, n_ref, w1_ref, b1_ref, w2_ref, b2_ref,
                w3_ref, b3_ref, out_ref):
    def head(pooled):
        h = jnp.maximum(
            jnp.dot(pooled, w1_ref[...], preferred_element_type=jnp.float32)
            + b1_ref[...], 0.0)
        h = jnp.maximum(
            jnp.dot(h, w2_ref[...], preferred_element_type=jnp.float32)
            + b2_ref[...], 0.0)
        return (jnp.dot(h, w3_ref[...], preferred_element_type=jnp.float32)
                + b3_ref[...])

    n = n_ref[...].astype(jnp.float32)
    out_ref[:K_SC] = head(sc_ref[...] / n[:K_SC])
    out_ref[K_SC:] = head(tc_ref[...] / n[K_SC:])


def _tc_mlp(sc_sums, tc_sums, n_node, W1, b1, W2, b2, W3, b3):
    return pl.pallas_call(
        _mlp_kernel,
        out_shape=jax.ShapeDtypeStruct((B_GRAPHS, OUT_DIM), jnp.float32),
    )(sc_sums, tc_sums, n_node.reshape(B_GRAPHS, 1), W1, b1, W2, b2, W3, b3)


@jax.jit
def kernel(feat, n_node, W1, b1, W2, b2, W3, b3):
    sc_sums = _sc_pool(feat.reshape(-1))
    tc_sums = _tc_pool(feat.reshape(B_GRAPHS, NPG, LATENT))
    return _tc_mlp(sc_sums, tc_sums, n_node, W1, b1, W2, b2, W3, b3)
